# Initial kernel scaffold; baseline (speedup 1.0000x reference)
import jax
import jax.numpy as jnp
from jax.experimental import pallas as pl


def kernel(x, thetas, qbias):
    raise NotImplementedError("write your pallas kernel here")


# speedup vs baseline: 6.8750x; 6.8750x over previous
import math
import numpy as np
import jax
import jax.numpy as jnp
from jax import lax
from jax.experimental import pallas as pl

_FS = 2
_C_IN = 3
_C_OUT = 8
_VQC = 3
_IDPC = _C_IN * _FS * _FS
_NQ = math.ceil(math.log2(_IDPC))
_DIM = 2 ** _NQ


def _const_mats():
    idx = np.arange(_DIM)
    qs = []
    for q in range(_NQ):
        b = 1 << (_NQ - 1 - q)
        Q = np.zeros((_DIM, _DIM), np.float32)
        for i in range(_DIM):
            Q[i ^ b, i] = 1.0 if (i & b) else -1.0
        qs.append(Q)
    P = np.eye(_DIM, dtype=np.float32)
    for q in range(_NQ):
        cbit = 1 << (_NQ - 1 - q)
        tbit = 1 << (_NQ - 1 - ((q + 1) % _NQ))
        perm = np.where((idx & cbit) > 0, idx ^ tbit, idx)
        Pq = np.zeros((_DIM, _DIM), np.float32)
        for i in range(_DIM):
            Pq[perm[i], i] = 1.0
        P = P @ Pq
    signs = np.where((idx & (1 << (_NQ - 1))) > 0, -1.0, 1.0).astype(np.float32)
    return qs, P, signs


_QS_NP, _PERM_NP, _SIGNS_NP = _const_mats()


def _prep_body(th_ref, qs_ref, p_ref, eye_ref, sg_ref, m_ref):
    th = th_ref[...]
    C = jnp.cos(th * 0.5)
    S = jnp.sin(th * 0.5)
    eye = eye_ref[...]
    qs = [qs_ref[q] for q in range(_NQ)]
    P = p_ref[...]
    sg = sg_ref[...]
    ms = []
    for o in range(_C_OUT):
        T = eye
        for l in range(_VQC):
            for q in range(_NQ):
                i = l * _NQ + q
                R = C[o:o + 1, i:i + 1] * eye + S[o:o + 1, i:i + 1] * qs[q]
                T = jnp.dot(T, R, preferred_element_type=jnp.float32,
                            precision=lax.Precision.HIGHEST)
            T = jnp.dot(T, P, preferred_element_type=jnp.float32,
                        precision=lax.Precision.HIGHEST)
        M = lax.dot_general(T * sg, T,
                            (((1,), (1,)), ((), ())),
                            preferred_element_type=jnp.float32,
                            precision=lax.Precision.HIGHEST)
        ms.append(M * (2.0 - eye))
    m_ref[...] = jnp.stack(ms, axis=0)


def _conv_body(m_ref, qb_ref, x_ref, xs_ref, o_ref):
    W = x_ref.shape[3]
    ow = W - _FS + 1
    slabs = []
    for c in range(_C_IN):
        for fi in range(_FS):
            src = x_ref if fi == 0 else xs_ref
            for fj in range(_FS):
                slabs.append(src[0, c, :, fj:fj + ow])
    norm2 = None
    accs = [None] * _C_OUT
    for k in range(_IDPC):
        for l in range(k, _IDPC):
            f = slabs[k] * slabs[l]
            if k == l:
                norm2 = f if norm2 is None else norm2 + f
            for o in range(_C_OUT):
                t = m_ref[o, k, l] * f
                accs[o] = t if accs[o] is None else accs[o] + t
    inv = jnp.where(norm2 > 0, 1.0 / norm2, jnp.zeros_like(norm2))
    for o in range(_C_OUT):
        o_ref[0, o] = accs[o] * inv + qb_ref[0, o]


def kernel(x, thetas, qbias):
    B, C, H, W = x.shape
    oh, ow = H - _FS + 1, W - _FS + 1
    th = thetas.reshape(_C_OUT, _VQC * _NQ)
    M = pl.pallas_call(
        _prep_body,
        out_shape=jax.ShapeDtypeStruct((_C_OUT, _DIM, _DIM), jnp.float32),
    )(th,
      jnp.asarray(np.stack(_QS_NP, axis=0)),
      jnp.asarray(_PERM_NP),
      jnp.asarray(np.eye(_DIM, dtype=np.float32)),
      jnp.asarray(_SIGNS_NP.reshape(1, _DIM)))
    qb = qbias.reshape(1, _C_OUT)
    xs = x[:, :, 1:, :]
    R = 32
    nr = (oh + R - 1) // R
    out = pl.pallas_call(
        _conv_body,
        grid=(B, nr),
        in_specs=[
            pl.BlockSpec((_C_OUT, _DIM, _DIM), lambda b, r: (0, 0, 0)),
            pl.BlockSpec((1, _C_OUT), lambda b, r: (0, 0)),
            pl.BlockSpec((1, C, R, W), lambda b, r: (b, 0, r, 0)),
            pl.BlockSpec((1, C, R, W), lambda b, r: (b, 0, r, 0)),
        ],
        out_specs=pl.BlockSpec((1, _C_OUT, R, ow), lambda b, r: (b, 0, r, 0)),
        out_shape=jax.ShapeDtypeStruct((B, _C_OUT, oh, ow), jnp.float32),
    )(M, qb, x, xs)
    return out


# speedup vs baseline: 74.3917x; 10.8206x over previous
import math
import numpy as np
import jax
import jax.numpy as jnp
from jax import lax
from jax.experimental import pallas as pl

_FS = 2
_C_IN = 3
_C_OUT = 8
_VQC = 3
_IDPC = _C_IN * _FS * _FS
_NQ = math.ceil(math.log2(_IDPC))
_DIM = 2 ** _NQ


def _const_mats():
    idx = np.arange(_DIM)
    qs = []
    for q in range(_NQ):
        b = 1 << (_NQ - 1 - q)
        Q = np.zeros((_DIM, _DIM), np.float32)
        for i in range(_DIM):
            Q[i ^ b, i] = 1.0 if (i & b) else -1.0
        qs.append(Q)
    P = np.eye(_DIM, dtype=np.float32)
    for q in range(_NQ):
        cbit = 1 << (_NQ - 1 - q)
        tbit = 1 << (_NQ - 1 - ((q + 1) % _NQ))
        perm = np.where((idx & cbit) > 0, idx ^ tbit, idx)
        Pq = np.zeros((_DIM, _DIM), np.float32)
        for i in range(_DIM):
            Pq[perm[i], i] = 1.0
        P = P @ Pq
    signs = np.where((idx & (1 << (_NQ - 1))) > 0, -1.0, 1.0).astype(np.float32)
    return qs, P, signs


_QS_NP, _PERM_NP, _SIGNS_NP = _const_mats()


def _prep_body(th_ref, qs_ref, p_ref, eye_ref, sg_ref, m_ref):
    th = th_ref[...]
    C = jnp.cos(th * 0.5)
    S = jnp.sin(th * 0.5)
    eye = eye_ref[...]
    qs = [qs_ref[q] for q in range(_NQ)]
    P = p_ref[...]
    sg = sg_ref[...]
    ms = []
    for o in range(_C_OUT):
        T = eye
        for l in range(_VQC):
            for q in range(_NQ):
                i = l * _NQ + q
                R = C[o:o + 1, i:i + 1] * eye + S[o:o + 1, i:i + 1] * qs[q]
                T = jnp.dot(T, R, preferred_element_type=jnp.float32,
                            precision=lax.Precision.HIGHEST)
            T = jnp.dot(T, P, preferred_element_type=jnp.float32,
                        precision=lax.Precision.HIGHEST)
        M = lax.dot_general(T * sg, T,
                            (((1,), (1,)), ((), ())),
                            preferred_element_type=jnp.float32,
                            precision=lax.Precision.HIGHEST)
        ms.append(M * (2.0 - eye))
    m_ref[...] = jnp.stack(ms, axis=0)


def _conv_body(m_ref, qb_ref, s_ref, o_ref):
    slabs = [s_ref[0, k] for k in range(_IDPC)]
    norm2 = None
    accs = [None] * _C_OUT
    for k in range(_IDPC):
        for l in range(k, _IDPC):
            f = slabs[k] * slabs[l]
            if k == l:
                norm2 = f if norm2 is None else norm2 + f
            for o in range(_C_OUT):
                t = m_ref[o, k, l] * f
                accs[o] = t if accs[o] is None else accs[o] + t
    inv = jnp.where(norm2 > 0, 1.0 / norm2, jnp.zeros_like(norm2))
    for o in range(_C_OUT):
        o_ref[0, o] = accs[o] * inv + qb_ref[0, o]


def kernel(x, thetas, qbias):
    B, C, H, W = x.shape
    oh, ow = H - _FS + 1, W - _FS + 1
    th = thetas.reshape(_C_OUT, _VQC * _NQ)
    M = pl.pallas_call(
        _prep_body,
        out_shape=jax.ShapeDtypeStruct((_C_OUT, _DIM, _DIM), jnp.float32),
    )(th,
      jnp.asarray(np.stack(_QS_NP, axis=0)),
      jnp.asarray(_PERM_NP),
      jnp.asarray(np.eye(_DIM, dtype=np.float32)),
      jnp.asarray(_SIGNS_NP.reshape(1, _DIM)))
    qb = qbias.reshape(1, _C_OUT)
    views = []
    for c in range(_C_IN):
        for fi in range(_FS):
            for fj in range(_FS):
                views.append(x[:, c, fi:fi + oh, fj:fj + ow])
    S = jnp.stack(views, axis=1)
    R = 32
    nr = (oh + R - 1) // R
    out = pl.pallas_call(
        _conv_body,
        grid=(B, nr),
        in_specs=[
            pl.BlockSpec((_C_OUT, _DIM, _DIM), lambda b, r: (0, 0, 0)),
            pl.BlockSpec((1, _C_OUT), lambda b, r: (0, 0)),
            pl.BlockSpec((1, _IDPC, R, ow), lambda b, r: (b, 0, r, 0)),
        ],
        out_specs=pl.BlockSpec((1, _C_OUT, R, ow), lambda b, r: (b, 0, r, 0)),
        out_shape=jax.ShapeDtypeStruct((B, _C_OUT, oh, ow), jnp.float32),
    )(M, qb, S)
    return out


# speedup vs baseline: 84.7517x; 1.1393x over previous
import math
import numpy as np
import jax
import jax.numpy as jnp
from jax import lax
from jax.experimental import pallas as pl
from jax.experimental.pallas import tpu as pltpu

_FS = 2
_C_IN = 3
_C_OUT = 8
_VQC = 3
_IDPC = _C_IN * _FS * _FS
_NQ = math.ceil(math.log2(_IDPC))
_DIM = 2 ** _NQ


def _const_mats():
    idx = np.arange(_DIM)
    qs = []
    for q in range(_NQ):
        b = 1 << (_NQ - 1 - q)
        Q = np.zeros((_DIM, _DIM), np.float32)
        for i in range(_DIM):
            Q[i ^ b, i] = 1.0 if (i & b) else -1.0
        qs.append(Q)
    P = np.eye(_DIM, dtype=np.float32)
    for q in range(_NQ):
        cbit = 1 << (_NQ - 1 - q)
        tbit = 1 << (_NQ - 1 - ((q + 1) % _NQ))
        perm = np.where((idx & cbit) > 0, idx ^ tbit, idx)
        Pq = np.zeros((_DIM, _DIM), np.float32)
        for i in range(_DIM):
            Pq[perm[i], i] = 1.0
        P = P @ Pq
    signs = np.where((idx & (1 << (_NQ - 1))) > 0, -1.0, 1.0).astype(np.float32)
    return qs, P, signs


_QS_NP, _PERM_NP, _SIGNS_NP = _const_mats()


def _prep_body(th_ref, qs_ref, p_ref, eye_ref, sg_ref, m_ref):
    th = th_ref[...]
    C = jnp.cos(th * 0.5)
    S = jnp.sin(th * 0.5)
    eye = eye_ref[...]
    qs = [qs_ref[q] for q in range(_NQ)]
    P = p_ref[...]
    sg = sg_ref[...]
    ms = []
    for o in range(_C_OUT):
        T = eye
        for l in range(_VQC):
            for q in range(_NQ):
                i = l * _NQ + q
                R = C[o:o + 1, i:i + 1] * eye + S[o:o + 1, i:i + 1] * qs[q]
                T = jnp.dot(T, R, preferred_element_type=jnp.float32,
                            precision=lax.Precision.HIGHEST)
            T = jnp.dot(T, P, preferred_element_type=jnp.float32,
                        precision=lax.Precision.HIGHEST)
        M = lax.dot_general(T * sg, T,
                            (((1,), (1,)), ((), ())),
                            preferred_element_type=jnp.float32,
                            precision=lax.Precision.HIGHEST)
        ms.append(M * (2.0 - eye))
    m_ref[...] = jnp.stack(ms, axis=0)


def _conv_body(m_ref, qb_ref, x_ref, xs_ref, o_ref, sc_ref):
    W = x_ref.shape[3]
    ow = W - _FS + 1
    for c in range(_C_IN):
        for fi in range(_FS):
            src = x_ref if fi == 0 else xs_ref
            for fj in range(_FS):
                sc_ref[c * _FS * _FS + fi * _FS + fj] = src[0, c, :, fj:fj + ow]
    slabs = [sc_ref[k] for k in range(_IDPC)]
    norm2 = None
    accs = [None] * _C_OUT
    for k in range(_IDPC):
        for l in range(k, _IDPC):
            f = slabs[k] * slabs[l]
            if k == l:
                norm2 = f if norm2 is None else norm2 + f
            for o in range(_C_OUT):
                t = m_ref[o, k, l] * f
                accs[o] = t if accs[o] is None else accs[o] + t
    inv = jnp.where(norm2 > 0, 1.0 / norm2, jnp.zeros_like(norm2))
    for o in range(_C_OUT):
        o_ref[0, o] = accs[o] * inv + qb_ref[0, o]


def kernel(x, thetas, qbias):
    B, C, H, W = x.shape
    oh, ow = H - _FS + 1, W - _FS + 1
    th = thetas.reshape(_C_OUT, _VQC * _NQ)
    M = pl.pallas_call(
        _prep_body,
        out_shape=jax.ShapeDtypeStruct((_C_OUT, _DIM, _DIM), jnp.float32),
    )(th,
      jnp.asarray(np.stack(_QS_NP, axis=0)),
      jnp.asarray(_PERM_NP),
      jnp.asarray(np.eye(_DIM, dtype=np.float32)),
      jnp.asarray(_SIGNS_NP.reshape(1, _DIM)))
    qb = qbias.reshape(1, _C_OUT)
    xs = x[:, :, 1:, :]
    R = 32
    nr = (oh + R - 1) // R
    out = pl.pallas_call(
        _conv_body,
        grid=(B, nr),
        in_specs=[
            pl.BlockSpec((_C_OUT, _DIM, _DIM), lambda b, r: (0, 0, 0)),
            pl.BlockSpec((1, _C_OUT), lambda b, r: (0, 0)),
            pl.BlockSpec((1, C, R, W), lambda b, r: (b, 0, r, 0)),
            pl.BlockSpec((1, C, R, W), lambda b, r: (b, 0, r, 0)),
        ],
        out_specs=pl.BlockSpec((1, _C_OUT, R, ow), lambda b, r: (b, 0, r, 0)),
        out_shape=jax.ShapeDtypeStruct((B, _C_OUT, oh, ow), jnp.float32),
        scratch_shapes=[pltpu.VMEM((_IDPC, R, ow), jnp.float32)],
    )(M, qb, x, xs)
    return out


# speedup vs baseline: 92.3698x; 1.0899x over previous
import math
import numpy as np
import jax
import jax.numpy as jnp
from jax import lax
from jax.experimental import pallas as pl
from jax.experimental.pallas import tpu as pltpu

_FS = 2
_C_IN = 3
_C_OUT = 8
_VQC = 3
_IDPC = _C_IN * _FS * _FS
_NQ = math.ceil(math.log2(_IDPC))
_DIM = 2 ** _NQ


def _const_mats():
    idx = np.arange(_DIM)
    qs = []
    for q in range(_NQ):
        b = 1 << (_NQ - 1 - q)
        Q = np.zeros((_DIM, _DIM), np.float32)
        for i in range(_DIM):
            Q[i ^ b, i] = 1.0 if (i & b) else -1.0
        qs.append(Q)
    P = np.eye(_DIM, dtype=np.float32)
    for q in range(_NQ):
        cbit = 1 << (_NQ - 1 - q)
        tbit = 1 << (_NQ - 1 - ((q + 1) % _NQ))
        perm = np.where((idx & cbit) > 0, idx ^ tbit, idx)
        Pq = np.zeros((_DIM, _DIM), np.float32)
        for i in range(_DIM):
            Pq[perm[i], i] = 1.0
        P = P @ Pq
    signs = np.where((idx & (1 << (_NQ - 1))) > 0, -1.0, 1.0).astype(np.float32)
    return qs, P, signs


_QS_NP, _PERM_NP, _SIGNS_NP = _const_mats()
_NB = _C_OUT * _DIM


def _blockdiag(A):
    Z = np.zeros((_NB, _NB), np.float32)
    for o in range(_C_OUT):
        Z[o * _DIM:(o + 1) * _DIM, o * _DIM:(o + 1) * _DIM] = A
    return Z


def _hdot(a, b):
    return jnp.dot(a, b, preferred_element_type=jnp.float32,
                   precision=lax.Precision.HIGHEST)


def _prep_body(th_ref, qs_ref, p_ref, eye_ref, sg_ref, rep_ref, dbl_ref,
               m_ref):
    th = th_ref[...]
    C = jnp.cos(th * 0.5)
    S = jnp.sin(th * 0.5)
    eye = eye_ref[...]
    P = p_ref[...]
    sg = sg_ref[...]
    rep = rep_ref[...]
    crep = _hdot(rep, C)
    srep = _hdot(rep, S)
    T = eye
    for l in range(_VQC):
        for q in range(_NQ):
            i = l * _NQ + q
            R = crep[:, i:i + 1] * eye + srep[:, i:i + 1] * qs_ref[q]
            T = _hdot(T, R)
        T = _hdot(T, P)
    M = lax.dot_general(T * sg, T,
                        (((1,), (1,)), ((), ())),
                        preferred_element_type=jnp.float32,
                        precision=lax.Precision.HIGHEST)
    m_ref[...] = M * dbl_ref[...]


def _conv_body(m_ref, qb_ref, x_ref, xs_ref, o_ref, sc_ref):
    W = x_ref.shape[3]
    ow = W - _FS + 1
    for c in range(_C_IN):
        for fi in range(_FS):
            src = x_ref if fi == 0 else xs_ref
            for fj in range(_FS):
                sc_ref[c * _FS * _FS + fi * _FS + fj] = src[0, c, :, fj:fj + ow]
    slabs = [sc_ref[k] for k in range(_IDPC)]
    norm2 = None
    accs = [None] * _C_OUT
    for k in range(_IDPC):
        for l in range(k, _IDPC):
            f = slabs[k] * slabs[l]
            if k == l:
                norm2 = f if norm2 is None else norm2 + f
            for o in range(_C_OUT):
                t = m_ref[_DIM * o + k, _DIM * o + l] * f
                accs[o] = t if accs[o] is None else accs[o] + t
    inv = jnp.where(norm2 > 0, 1.0 / norm2, jnp.zeros_like(norm2))
    for o in range(_C_OUT):
        o_ref[0, o] = accs[o] * inv + qb_ref[0, o]


def kernel(x, thetas, qbias):
    B, C, H, W = x.shape
    oh, ow = H - _FS + 1, W - _FS + 1
    th = thetas.reshape(_C_OUT, _VQC * _NQ)
    rep = np.zeros((_NB, _C_OUT), np.float32)
    for i in range(_NB):
        rep[i, i // _DIM] = 1.0
    M = pl.pallas_call(
        _prep_body,
        out_shape=jax.ShapeDtypeStruct((_NB, _NB), jnp.float32),
    )(th,
      jnp.asarray(np.stack([_blockdiag(Q) for Q in _QS_NP], axis=0)),
      jnp.asarray(_blockdiag(_PERM_NP)),
      jnp.asarray(np.eye(_NB, dtype=np.float32)),
      jnp.asarray(np.tile(_SIGNS_NP, _C_OUT).reshape(1, _NB)),
      jnp.asarray(rep),
      jnp.asarray(2.0 - np.eye(_NB, dtype=np.float32)))
    qb = qbias.reshape(1, _C_OUT)
    xs = x[:, :, 1:, :]
    R = 32
    nr = (oh + R - 1) // R
    out = pl.pallas_call(
        _conv_body,
        grid=(B, nr),
        in_specs=[
            pl.BlockSpec((_NB, _NB), lambda b, r: (0, 0)),
            pl.BlockSpec((1, _C_OUT), lambda b, r: (0, 0)),
            pl.BlockSpec((1, C, R, W), lambda b, r: (b, 0, r, 0)),
            pl.BlockSpec((1, C, R, W), lambda b, r: (b, 0, r, 0)),
        ],
        out_specs=pl.BlockSpec((1, _C_OUT, R, ow), lambda b, r: (b, 0, r, 0)),
        out_shape=jax.ShapeDtypeStruct((B, _C_OUT, oh, ow), jnp.float32),
        scratch_shapes=[pltpu.VMEM((_IDPC, R, ow), jnp.float32)],
    )(M, qb, x, xs)
    return out
